# Initial kernel scaffold; baseline (speedup 1.0000x reference)
import jax
import jax.numpy as jnp
from jax.experimental import pallas as pl


def kernel(x, day_indicies, W1, b1, W2, b2, gamma, beta):
    raise NotImplementedError("write your pallas kernel here")


# speedup vs baseline: 2.7118x; 2.7118x over previous
import jax
import jax.numpy as jnp
from jax.experimental import pallas as pl
from jax.experimental.pallas import tpu as pltpu

EPS = 1e-5


def _body(sdays_ref, perm_ref, x_ref, W1_ref, b1_ref, W2_ref, b2_ref,
          g_ref, be_ref, out_ref):
    xb = x_ref[0]
    h = jnp.dot(xb, W1_ref[0], preferred_element_type=jnp.float32)
    h = jnp.maximum(h + b1_ref[0], 0.0)
    y = jnp.dot(h, W2_ref[0], preferred_element_type=jnp.float32)
    y = y + b2_ref[0]
    mu = jnp.mean(y, axis=-1, keepdims=True)
    yc = y - mu
    var = jnp.mean(yc * yc, axis=-1, keepdims=True)
    out_ref[0] = yc * jax.lax.rsqrt(var + EPS) * g_ref[0] + be_ref[0]


def kernel(x, day_indicies, W1, b1, W2, b2, gamma, beta):
    B, T, IN = x.shape
    D, _, HID = W1.shape
    OUT = W2.shape[2]

    day = day_indicies.astype(jnp.int32)
    perm = jnp.argsort(day).astype(jnp.int32)
    sdays = jnp.take(day, perm)

    b1r = b1.reshape(D, 1, HID)
    b2r = b2.reshape(D, 1, OUT)
    gr = gamma.reshape(D, 1, OUT)
    br = beta.reshape(D, 1, OUT)

    grid_spec = pltpu.PrefetchScalarGridSpec(
        num_scalar_prefetch=2,
        grid=(B,),
        in_specs=[
            pl.BlockSpec((1, T, IN), lambda i, sd, pm: (pm[i], 0, 0)),
            pl.BlockSpec((1, IN, HID), lambda i, sd, pm: (sd[i], 0, 0)),
            pl.BlockSpec((1, 1, HID), lambda i, sd, pm: (sd[i], 0, 0)),
            pl.BlockSpec((1, HID, OUT), lambda i, sd, pm: (sd[i], 0, 0)),
            pl.BlockSpec((1, 1, OUT), lambda i, sd, pm: (sd[i], 0, 0)),
            pl.BlockSpec((1, 1, OUT), lambda i, sd, pm: (sd[i], 0, 0)),
            pl.BlockSpec((1, 1, OUT), lambda i, sd, pm: (sd[i], 0, 0)),
        ],
        out_specs=pl.BlockSpec((1, T, OUT), lambda i, sd, pm: (pm[i], 0, 0)),
    )

    return pl.pallas_call(
        _body,
        grid_spec=grid_spec,
        out_shape=jax.ShapeDtypeStruct((B, T, OUT), jnp.float32),
        compiler_params=pltpu.CompilerParams(
            dimension_semantics=("arbitrary",),
        ),
    )(sdays, perm, x, W1, b1r, W2, b2r, gr, br)


# speedup vs baseline: 2.7131x; 1.0005x over previous
import jax
import jax.numpy as jnp
from jax.experimental import pallas as pl
from jax.experimental.pallas import tpu as pltpu

EPS = 1e-5


def _body(sdays_ref, perm_ref, x_ref, W1_ref, b1_ref, W2_ref, b2_ref,
          g_ref, be_ref, out_ref):
    xb = x_ref[0].astype(jnp.bfloat16)
    h = jnp.dot(xb, W1_ref[0].astype(jnp.bfloat16),
                preferred_element_type=jnp.float32)
    h = jnp.maximum(h + b1_ref[0], 0.0).astype(jnp.bfloat16)
    y = jnp.dot(h, W2_ref[0].astype(jnp.bfloat16),
                preferred_element_type=jnp.float32)
    y = y + b2_ref[0]
    mu = jnp.mean(y, axis=-1, keepdims=True)
    yc = y - mu
    var = jnp.mean(yc * yc, axis=-1, keepdims=True)
    out_ref[0] = yc * jax.lax.rsqrt(var + EPS) * g_ref[0] + be_ref[0]


def kernel(x, day_indicies, W1, b1, W2, b2, gamma, beta):
    B, T, IN = x.shape
    D, _, HID = W1.shape
    OUT = W2.shape[2]

    day = day_indicies.astype(jnp.int32)
    perm = jnp.argsort(day).astype(jnp.int32)
    sdays = jnp.take(day, perm)

    b1r = b1.reshape(D, 1, HID)
    b2r = b2.reshape(D, 1, OUT)
    gr = gamma.reshape(D, 1, OUT)
    br = beta.reshape(D, 1, OUT)

    grid_spec = pltpu.PrefetchScalarGridSpec(
        num_scalar_prefetch=2,
        grid=(B,),
        in_specs=[
            pl.BlockSpec((1, T, IN), lambda i, sd, pm: (pm[i], 0, 0)),
            pl.BlockSpec((1, IN, HID), lambda i, sd, pm: (sd[i], 0, 0)),
            pl.BlockSpec((1, 1, HID), lambda i, sd, pm: (sd[i], 0, 0)),
            pl.BlockSpec((1, HID, OUT), lambda i, sd, pm: (sd[i], 0, 0)),
            pl.BlockSpec((1, 1, OUT), lambda i, sd, pm: (sd[i], 0, 0)),
            pl.BlockSpec((1, 1, OUT), lambda i, sd, pm: (sd[i], 0, 0)),
            pl.BlockSpec((1, 1, OUT), lambda i, sd, pm: (sd[i], 0, 0)),
        ],
        out_specs=pl.BlockSpec((1, T, OUT), lambda i, sd, pm: (pm[i], 0, 0)),
    )

    return pl.pallas_call(
        _body,
        grid_spec=grid_spec,
        out_shape=jax.ShapeDtypeStruct((B, T, OUT), jnp.float32),
        compiler_params=pltpu.CompilerParams(
            dimension_semantics=("arbitrary",),
        ),
    )(sdays, perm, x, W1, b1r, W2, b2r, gr, br)
